# baseline (speedup 1.0000x reference)
import functools

import jax
import jax.numpy as jnp
from jax import lax
from jax.experimental import pallas as pl
from jax.experimental.pallas import tpu as pltpu
from jax.experimental.pallas import tpu_sc as plsc

_CHUNK_ROWS = 4
_NBUF = 14


def _make_sc_copy(rows: int, d_model: int, dtype):
    info = plsc.get_sparse_core_info()
    nw = info.num_cores * info.num_subcores
    rows_per_w = rows // nw
    ch = min(_CHUNK_ROWS, rows_per_w)
    nchunks = rows_per_w // ch
    nbuf = min(_NBUF, nchunks)
    mesh = plsc.VectorSubcoreMesh(core_axis_name="c", subcore_axis_name="s")

    @functools.partial(
        pl.kernel,
        mesh=mesh,
        out_type=jax.ShapeDtypeStruct((rows, d_model), dtype),
        scratch_types=[
            pltpu.VMEM((nbuf, ch, d_model), dtype),
            pltpu.SemaphoreType.DMA((nbuf,)),
            pltpu.SemaphoreType.DMA((nbuf,)),
        ],
    )
    def copy_k(w_hbm, out_hbm, buf, in_sems, out_sems):
        wid = lax.axis_index("s") * info.num_cores + lax.axis_index("c")
        base = wid * rows_per_w

        def in_cp(i, s):
            return pltpu.make_async_copy(
                w_hbm.at[pl.ds(base + i * ch, ch)], buf.at[s], in_sems.at[s]
            )

        def out_cp(i, s):
            return pltpu.make_async_copy(
                buf.at[s], out_hbm.at[pl.ds(base + i * ch, ch)], out_sems.at[s]
            )

        for b in range(nbuf):
            in_cp(b, b).start()

        def body(i, carry):
            s = lax.rem(i, nbuf)
            in_cp(i, s).wait()
            out_cp(i, s).start()

            @pl.when(i + nbuf < nchunks)
            def _():
                out_cp(i, s).wait()
                in_cp(i + nbuf, s).start()

            return carry

        lax.fori_loop(0, nchunks, body, 0)
        for i in range(max(0, nchunks - nbuf), nchunks):
            out_cp(i, i % nbuf).wait()

    return copy_k


def kernel(x, W):
    b, seq_len = x.shape
    out = _make_sc_copy(seq_len, W.shape[1], W.dtype)(W)
    return out[None]


# speedup vs baseline: 1.0167x; 1.0167x over previous
import functools

import jax
import jax.numpy as jnp
from jax import lax
from jax.experimental import pallas as pl
from jax.experimental.pallas import tpu as pltpu
from jax.experimental.pallas import tpu_sc as plsc

_CHUNK_ROWS = 8
_NBUF = 7


def _make_sc_copy(rows: int, d_model: int, dtype):
    info = plsc.get_sparse_core_info()
    nw = info.num_cores * info.num_subcores
    rows_per_w = rows // nw
    ch = min(_CHUNK_ROWS, rows_per_w)
    nchunks = rows_per_w // ch
    nbuf = min(_NBUF, nchunks)
    mesh = plsc.VectorSubcoreMesh(core_axis_name="c", subcore_axis_name="s")

    @functools.partial(
        pl.kernel,
        mesh=mesh,
        out_type=jax.ShapeDtypeStruct((rows, d_model), dtype),
        scratch_types=[
            pltpu.VMEM((nbuf, ch, d_model), dtype),
            pltpu.SemaphoreType.DMA((nbuf,)),
            pltpu.SemaphoreType.DMA((nbuf,)),
        ],
    )
    def copy_k(w_hbm, out_hbm, buf, in_sems, out_sems):
        wid = lax.axis_index("s") * info.num_cores + lax.axis_index("c")
        base = wid * rows_per_w

        def in_cp(i, s):
            return pltpu.make_async_copy(
                w_hbm.at[pl.ds(base + i * ch, ch)], buf.at[s], in_sems.at[s]
            )

        def out_cp(i, s):
            return pltpu.make_async_copy(
                buf.at[s], out_hbm.at[pl.ds(base + i * ch, ch)], out_sems.at[s]
            )

        for b in range(nbuf):
            in_cp(b, b).start()

        def body(i, carry):
            s = lax.rem(i, nbuf)
            in_cp(i, s).wait()
            out_cp(i, s).start()

            @pl.when(i + nbuf < nchunks)
            def _():
                out_cp(i, s).wait()
                in_cp(i + nbuf, s).start()

            return carry

        lax.fori_loop(0, nchunks, body, 0)
        for i in range(max(0, nchunks - nbuf), nchunks):
            out_cp(i, i % nbuf).wait()

    return copy_k


def kernel(x, W):
    b, seq_len = x.shape
    out = _make_sc_copy(seq_len, W.shape[1], W.dtype)(W)
    return out[None]
